# baseline (speedup 1.0000x reference)
import functools

import jax
import jax.numpy as jnp
from jax import lax
from jax.experimental import pallas as pl
from jax.experimental.pallas import tpu as pltpu
from jax.experimental.pallas import tpu_sc as plsc

N_U = 30000
N_I = 20000
NN = 50000
EE = 800000
DD = 64
BB = 4096

H = 25000
CH = 1568
HP = 16 * CH
K = 64
NB = 6
BLK = NB * K
NBLK = 131
PER_SUB = NBLK * BLK
EP = 16 * PER_SUB

_mesh = plsc.VectorSubcoreMesh(
    core_axis_name="c", subcore_axis_name="s", num_cores=2, num_subcores=16)


def _f32(shape):
    return jax.ShapeDtypeStruct(shape, jnp.float32)


@functools.partial(
    pl.kernel,
    out_type=_f32((NN, 16)),
    mesh=_mesh,
    compiler_params=pltpu.CompilerParams(use_tc_tiling_on_sc=False, needs_layout_passes=False),
    scratch_types=[
        pltpu.VMEM_SHARED((HP, 16), jnp.float32),
        pltpu.VMEM((2, BLK), jnp.int32),
        pltpu.VMEM((2, NB, K), jnp.int32),
        pltpu.VMEM((K, 16), jnp.float32),
        pltpu.SemaphoreType.DMA((2,)),
        pltpu.SemaphoreType.DMA((NB,)),
    ],
)
def _deg_kernel(rowp, zrs16, deg16, dacc, rowb, srow, onesv, isem, ssem):
    c = lax.axis_index("c")
    s = lax.axis_index("s")
    c_lo = c * H
    ebase = s * PER_SUB
    iota = lax.iota(jnp.int32, 16)
    pltpu.sync_copy(zrs16.at[pl.ds(s * CH, CH)], dacc.at[pl.ds(s * CH, CH)])
    pat = jnp.where(iota == 0, jnp.float32(1.0), jnp.float32(0.0))
    for k in range(K):
        onesv[k, :] = pat
    plsc.subcore_barrier()

    pltpu.async_copy(rowp.at[pl.ds(ebase, BLK)], rowb.at[0], isem.at[0])

    def block(b, nch_prev):
        p = jnp.bitwise_and(b, 1)
        pltpu.make_async_copy(
            rowp.at[pl.ds(ebase + b * BLK, BLK)], rowb.at[p], isem.at[p]
        ).wait()
        @pl.when(b + 1 < NBLK)
        def _():
            pltpu.async_copy(rowp.at[pl.ds(ebase + (b + 1) * BLK, BLK)],
                             rowb.at[1 - p], isem.at[1 - p])
        for j in range(NB):
            @pl.when(j < nch_prev)
            def _():
                pltpu.make_async_copy(
                    onesv, dacc.at[srow.at[1 - p, j]], ssem.at[j]).wait()
        wpos = jnp.int32(0)
        for j in range(BLK // 16):
            r = rowb[p, pl.ds(j * 16, 16)]
            ok = (r >= c_lo) & (r < c_lo + H)
            oki = jnp.where(ok, jnp.int32(1), jnp.int32(0))
            off = wpos + plsc.cumsum(oki) - 1
            d0 = lax.shift_right_logical(off, 6)
            d1 = jnp.bitwise_and(off, 63)
            plsc.store_scatter(srow.at[p], [d0, d1], r - c_lo, mask=ok)
            wpos = wpos + lax.reduce_sum(oki, axes=(0,))
        nch = lax.shift_right_logical(wpos + (K - 1), 6)
        lim = nch * K
        for j in range(4):
            idxs = wpos + j * 16 + iota
            m = idxs < lim
            d0 = lax.shift_right_logical(idxs, 6)
            d1 = jnp.bitwise_and(idxs, 63)
            plsc.store_scatter(srow.at[p], [d0, d1],
                               jnp.full((16,), H, jnp.int32), mask=m)
        for j in range(NB):
            @pl.when(j < nch)
            def _():
                pltpu.async_copy(onesv, dacc.at[srow.at[p, j]],
                                 ssem.at[j], add=True)
        return nch

    nch_last = lax.fori_loop(0, NBLK, block, jnp.int32(0))
    lastp = (NBLK - 1) & 1
    for j in range(NB):
        @pl.when(j < nch_last)
        def _():
            pltpu.make_async_copy(
                onesv, dacc.at[srow.at[lastp, j]], ssem.at[j]).wait()
    plsc.subcore_barrier()
    ob = jnp.minimum(s * CH, H - CH)
    pltpu.sync_copy(dacc.at[pl.ds(ob, CH)], deg16.at[pl.ds(c_lo + ob, CH)])


@functools.partial(
    pl.kernel,
    out_type=_f32((NN, DD)),
    mesh=_mesh,
    compiler_params=pltpu.CompilerParams(use_tc_tiling_on_sc=False, needs_layout_passes=False),
    scratch_types=[
        pltpu.VMEM_SHARED((HP, DD), jnp.float32),
        pltpu.VMEM((2, BLK), jnp.int32),
        pltpu.VMEM((2, BLK), jnp.int32),
        pltpu.VMEM((2, NB, K), jnp.int32),
        pltpu.VMEM((2, NB, K), jnp.int32),
        pltpu.VMEM((NB, K, DD), jnp.float32),
        pltpu.SemaphoreType.DMA((2,)),
        pltpu.SemaphoreType.DMA((NB,)),
        pltpu.SemaphoreType.DMA((NB,)),
    ],
)
def _spmm_kernel(y, colp, rowp, zrs, out,
                 acc, colb, rowb, scol, srow, gbuf, isem, gsem, ssem):
    c = lax.axis_index("c")
    s = lax.axis_index("s")
    c_lo = c * H
    ebase = s * PER_SUB
    iota = lax.iota(jnp.int32, 16)
    pltpu.sync_copy(zrs.at[pl.ds(s * CH, CH)], acc.at[pl.ds(s * CH, CH)])
    plsc.subcore_barrier()

    pltpu.async_copy(colp.at[pl.ds(ebase, BLK)], colb.at[0], isem.at[0])
    pltpu.async_copy(rowp.at[pl.ds(ebase, BLK)], rowb.at[0], isem.at[0])

    def block(b, nch_prev):
        p = jnp.bitwise_and(b, 1)
        base = ebase + b * BLK
        pltpu.make_async_copy(
            colp.at[pl.ds(base, BLK)], colb.at[p], isem.at[p]).wait()
        pltpu.make_async_copy(
            rowp.at[pl.ds(base, BLK)], rowb.at[p], isem.at[p]).wait()
        @pl.when(b + 1 < NBLK)
        def _():
            nbase = base + BLK
            pltpu.async_copy(colp.at[pl.ds(nbase, BLK)],
                             colb.at[1 - p], isem.at[1 - p])
            pltpu.async_copy(rowp.at[pl.ds(nbase, BLK)],
                             rowb.at[1 - p], isem.at[1 - p])
        for j in range(NB):
            @pl.when(j < nch_prev)
            def _():
                pltpu.make_async_copy(
                    gbuf.at[j], acc.at[srow.at[1 - p, j]], ssem.at[j]).wait()
        wpos = jnp.int32(0)
        for j in range(BLK // 16):
            r = rowb[p, pl.ds(j * 16, 16)]
            cv = colb[p, pl.ds(j * 16, 16)]
            ok = (r >= c_lo) & (r < c_lo + H)
            oki = jnp.where(ok, jnp.int32(1), jnp.int32(0))
            off = wpos + plsc.cumsum(oki) - 1
            d0 = lax.shift_right_logical(off, 6)
            d1 = jnp.bitwise_and(off, 63)
            plsc.store_scatter(srow.at[p], [d0, d1], r - c_lo, mask=ok)
            plsc.store_scatter(scol.at[p], [d0, d1], cv, mask=ok)
            wpos = wpos + lax.reduce_sum(oki, axes=(0,))
        nch = lax.shift_right_logical(wpos + (K - 1), 6)
        lim = nch * K
        for j in range(4):
            idxs = wpos + j * 16 + iota
            m = idxs < lim
            d0 = lax.shift_right_logical(idxs, 6)
            d1 = jnp.bitwise_and(idxs, 63)
            plsc.store_scatter(srow.at[p], [d0, d1],
                               jnp.full((16,), H, jnp.int32), mask=m)
            plsc.store_scatter(scol.at[p], [d0, d1],
                               jnp.zeros((16,), jnp.int32), mask=m)
        for j in range(NB):
            @pl.when(j < nch)
            def _():
                pltpu.async_copy(y.at[scol.at[p, j]], gbuf.at[j], gsem.at[j])
        for j in range(NB):
            @pl.when(j < nch)
            def _():
                pltpu.make_async_copy(
                    y.at[scol.at[p, j]], gbuf.at[j], gsem.at[j]).wait()
                pltpu.async_copy(gbuf.at[j], acc.at[srow.at[p, j]],
                                 ssem.at[j], add=True)
        return nch

    nch_last = lax.fori_loop(0, NBLK, block, jnp.int32(0))
    lastp = (NBLK - 1) & 1
    for j in range(NB):
        @pl.when(j < nch_last)
        def _():
            pltpu.make_async_copy(
                gbuf.at[j], acc.at[srow.at[lastp, j]], ssem.at[j]).wait()
    plsc.subcore_barrier()
    ob = jnp.minimum(s * CH, H - CH)
    pltpu.sync_copy(acc.at[pl.ds(ob, CH)], out.at[pl.ds(c_lo + ob, CH)])


@functools.partial(
    pl.kernel,
    out_type=(_f32((BB, DD)), _f32((BB, DD)), _f32((BB, DD))),
    mesh=_mesh,
    compiler_params=pltpu.CompilerParams(use_tc_tiling_on_sc=False, needs_layout_passes=False),
    scratch_types=[
        pltpu.VMEM((128,), jnp.int32),
        pltpu.VMEM((128, DD), jnp.float32),
        pltpu.SemaphoreType.DMA,
    ],
)
def _bpr_gather_kernel(S, users, items, negs, anc, pos, neg, idxv, buf, sem):
    c = lax.axis_index("c")
    s = lax.axis_index("s")
    base = (s * 2 + c) * 128

    pltpu.sync_copy(users.at[pl.ds(base, 128)], idxv)
    pltpu.async_copy(S.at[idxv], buf, sem).wait()
    pltpu.sync_copy(buf, anc.at[pl.ds(base, 128)])

    for src, dst in ((items, pos), (negs, neg)):
        pltpu.sync_copy(src.at[pl.ds(base, 128)], idxv)
        for j in range(8):
            idxv[pl.ds(j * 16, 16)] = idxv[pl.ds(j * 16, 16)] + N_U
        pltpu.async_copy(S.at[idxv], buf, sem).wait()
        pltpu.sync_copy(buf, dst.at[pl.ds(base, 128)])


_RB = 2000


def _prep_body(deg_ref, x0_ref, dis_ref, y0_ref):
    d = jnp.clip(lax.rsqrt(deg_ref[:, 0:1] + 1e-6), 0.0, 10.0)
    dis_ref[...] = d
    y0_ref[...] = d * x0_ref[...]


_prep = pl.pallas_call(
    _prep_body,
    grid=(NN // _RB,),
    in_specs=[
        pl.BlockSpec((_RB, 16), lambda i: (i, 0)),
        pl.BlockSpec((_RB, DD), lambda i: (i, 0)),
    ],
    out_specs=[
        pl.BlockSpec((_RB, 1), lambda i: (i, 0)),
        pl.BlockSpec((_RB, DD), lambda i: (i, 0)),
    ],
    out_shape=[_f32((NN, 1)), _f32((NN, DD))],
)


def _scale_body(acc_ref, dis_ref, s_ref, snew_ref, y_ref):
    d = dis_ref[...]
    da = d * acc_ref[...]
    snew_ref[...] = s_ref[...] + da
    y_ref[...] = d * da


_scale = pl.pallas_call(
    _scale_body,
    grid=(NN // _RB,),
    in_specs=[
        pl.BlockSpec((_RB, DD), lambda i: (i, 0)),
        pl.BlockSpec((_RB, 1), lambda i: (i, 0)),
        pl.BlockSpec((_RB, DD), lambda i: (i, 0)),
    ],
    out_specs=[
        pl.BlockSpec((_RB, DD), lambda i: (i, 0)),
        pl.BlockSpec((_RB, DD), lambda i: (i, 0)),
    ],
    out_shape=[_f32((NN, DD)), _f32((NN, DD))],
)


def _loss_body(a_ref, p_ref, n_ref, o_ref):
    a = a_ref[...]
    p = p_ref[...]
    n = n_ref[...]
    diff = jnp.sum(a * p, axis=-1) - jnp.sum(a * n, axis=-1)
    bpr = -jnp.sum(jnp.log(jax.nn.sigmoid(diff) + 1e-12)) / float(BB)
    reg = 0.5 * (jnp.sum(a * a) + jnp.sum(p * p) + jnp.sum(n * n)) / float(BB)
    o_ref[...] = (bpr + reg).reshape(1, 1)


_loss = pl.pallas_call(
    _loss_body,
    out_shape=_f32((1, 1)),
)


def kernel(user_emb, item_emb, edge_index, users, items, neg_items):
    x0 = jnp.concatenate([user_emb, item_emb], axis=0)
    pad = EP - EE
    colp = jnp.concatenate([edge_index[1], jnp.zeros((pad,), jnp.int32)])
    rowp = jnp.concatenate([edge_index[0], jnp.full((pad,), -1, jnp.int32)])
    zrs = jnp.zeros((HP, DD), jnp.float32)
    zrs16 = jnp.zeros((HP, 16), jnp.float32)

    deg16 = _deg_kernel(rowp, zrs16)
    dis, y = _prep(deg16, x0)
    s_sum = x0
    for _ in range(3):
        acc = _spmm_kernel(y, colp, rowp, zrs)
        s_sum, y = _scale(acc, dis, s_sum)
    anc, pos, neg = _bpr_gather_kernel(s_sum, users, items, neg_items)
    return _loss(anc, pos, neg)[0, 0]


# speedup vs baseline: 5.7842x; 5.7842x over previous
import functools

import jax
import jax.numpy as jnp
from jax import lax
from jax.experimental import pallas as pl
from jax.experimental.pallas import tpu as pltpu
from jax.experimental.pallas import tpu_sc as plsc

N_U = 30000
N_I = 20000
NN = 50000
EE = 800000
DD = 64
BB = 4096

H = 25000
CH = 1568
HP = 16 * CH
K = 64
NB = 6
BLK = NB * K
NBLK = 131
PER_SUB = NBLK * BLK
EP = 16 * PER_SUB

CAP = 50688
CAPC = CAP // K
NCHT = 32 * CAPC
NSTG = 8

_mesh = plsc.VectorSubcoreMesh(
    core_axis_name="c", subcore_axis_name="s", num_cores=2, num_subcores=16)


def _f32(shape):
    return jax.ShapeDtypeStruct(shape, jnp.float32)


def _i32(shape):
    return jax.ShapeDtypeStruct(shape, jnp.int32)


@functools.partial(
    pl.kernel,
    out_type=(_i32((NCHT, K)), _i32((NCHT, K)), _i32((32, 16))),
    mesh=_mesh,
    compiler_params=pltpu.CompilerParams(use_tc_tiling_on_sc=False,
                                         needs_layout_passes=False),
    scratch_types=[
        pltpu.VMEM((2, BLK), jnp.int32),
        pltpu.VMEM((2, BLK), jnp.int32),
        pltpu.VMEM((2, NSTG, K), jnp.int32),
        pltpu.VMEM((2, NSTG, K), jnp.int32),
        pltpu.VMEM((16,), jnp.int32),
        pltpu.SemaphoreType.DMA((2,)),
        pltpu.SemaphoreType.DMA((NSTG,)),
    ],
)
def _part_kernel(colp, rowp, srowP, scolP, cnts, colb, rowb, sstg, cstg,
                 cntv, isem, fsem):
    c = lax.axis_index("c")
    s = lax.axis_index("s")
    w = c * 16 + s
    obase = w * CAPC
    c_lo = c * H
    ebase = s * PER_SUB
    iota = lax.iota(jnp.int32, 16)

    pltpu.async_copy(colp.at[pl.ds(ebase, BLK)], colb.at[0], isem.at[0])
    pltpu.async_copy(rowp.at[pl.ds(ebase, BLK)], rowb.at[0], isem.at[0])

    def block(b, carry):
        wpos, ocnt, nfl_prev = carry
        p = jnp.bitwise_and(b, 1)
        base = ebase + b * BLK
        pltpu.make_async_copy(
            colp.at[pl.ds(base, BLK)], colb.at[p], isem.at[p]).wait()
        pltpu.make_async_copy(
            rowp.at[pl.ds(base, BLK)], rowb.at[p], isem.at[p]).wait()

        @pl.when(b + 1 < NBLK)
        def _():
            nbase = base + BLK
            pltpu.async_copy(colp.at[pl.ds(nbase, BLK)],
                             colb.at[1 - p], isem.at[1 - p])
            pltpu.async_copy(rowp.at[pl.ds(nbase, BLK)],
                             rowb.at[1 - p], isem.at[1 - p])
        pch = obase + ocnt - nfl_prev
        for j in range(NSTG - 1):
            @pl.when(j < nfl_prev)
            def _():
                pltpu.make_async_copy(
                    sstg.at[1 - p, j], srowP.at[pch + j], fsem.at[j]).wait()
                pltpu.make_async_copy(
                    cstg.at[1 - p, j], scolP.at[pch + j], fsem.at[j]).wait()
        for j in range(BLK // 16):
            r = rowb[p, pl.ds(j * 16, 16)]
            cv = colb[p, pl.ds(j * 16, 16)]
            ok = (r >= c_lo) & (r < c_lo + H)
            oki = jnp.where(ok, jnp.int32(1), jnp.int32(0))
            off = wpos + plsc.cumsum(oki) - 1
            d0 = lax.shift_right_logical(off, 6)
            d1 = jnp.bitwise_and(off, 63)
            plsc.store_scatter(sstg.at[p], [d0, d1], r - c_lo, mask=ok)
            plsc.store_scatter(cstg.at[p], [d0, d1], cv, mask=ok)
            wpos = wpos + lax.reduce_sum(oki, axes=(0,))
        nfl = lax.shift_right_logical(wpos, 6)
        fch = obase + ocnt
        for j in range(NSTG - 1):
            @pl.when(j < nfl)
            def _():
                pltpu.async_copy(sstg.at[p, j], srowP.at[fch + j], fsem.at[j])
                pltpu.async_copy(cstg.at[p, j], scolP.at[fch + j], fsem.at[j])
        for i in range(4):
            sl = pl.ds(i * 16, 16)
            sstg[1 - p, 0, sl] = sstg[p, nfl, sl]
            cstg[1 - p, 0, sl] = cstg[p, nfl, sl]
        return jnp.bitwise_and(wpos, 63), ocnt + nfl, nfl

    wpos, ocnt, nfl_last = lax.fori_loop(
        0, NBLK, block, (jnp.int32(0), jnp.int32(0), jnp.int32(0)))
    lastp = (NBLK - 1) & 1
    pch = obase + ocnt - nfl_last
    for j in range(NSTG - 1):
        @pl.when(j < nfl_last)
        def _():
            pltpu.make_async_copy(
                sstg.at[1 - lastp, j], srowP.at[pch + j], fsem.at[j]).wait()
            pltpu.make_async_copy(
                cstg.at[1 - lastp, j], scolP.at[pch + j], fsem.at[j]).wait()
    fp = 1 - lastp
    for i in range(4):
        idxs = i * 16 + iota
        m = idxs >= wpos
        sl = pl.ds(i * 16, 16)
        plsc.store_scatter(sstg.at[fp, 0], [idxs], jnp.full((16,), H,
                                                           jnp.int32), mask=m)
        plsc.store_scatter(cstg.at[fp, 0], [idxs], jnp.zeros((16,),
                                                            jnp.int32), mask=m)
        sstg[fp, 1, sl] = jnp.full((16,), H, jnp.int32)
        cstg[fp, 1, sl] = jnp.zeros((16,), jnp.int32)
    pltpu.sync_copy(sstg.at[fp, 0], srowP.at[obase + ocnt])
    pltpu.sync_copy(cstg.at[fp, 0], scolP.at[obase + ocnt])
    ocnt = ocnt + 1
    nblk6 = lax.shift_right_logical(ocnt * 43691, 18)
    npad = nblk6 * 6 + 6 - ocnt
    npad = jnp.where(npad == 6, 0, npad)
    for j in range(5):
        @pl.when(j < npad)
        def _():
            pb = obase + ocnt + j
            pltpu.sync_copy(sstg.at[fp, 1], srowP.at[pb])
            pltpu.sync_copy(cstg.at[fp, 1], scolP.at[pb])
    ocnt = ocnt + npad
    nblk = lax.shift_right_logical(ocnt * 43691, 18)
    cntv[...] = jnp.full((16,), 1, jnp.int32) * nblk
    pltpu.sync_copy(cntv, cnts.at[w])


@functools.partial(
    pl.kernel,
    out_type=_f32((NN, 16)),
    mesh=_mesh,
    compiler_params=pltpu.CompilerParams(use_tc_tiling_on_sc=False, needs_layout_passes=False),
    scratch_types=[
        pltpu.VMEM_SHARED((HP, 16), jnp.float32),
        pltpu.VMEM((2, BLK), jnp.int32),
        pltpu.VMEM((2, NB, K), jnp.int32),
        pltpu.VMEM((K, 16), jnp.float32),
        pltpu.SemaphoreType.DMA((2,)),
        pltpu.SemaphoreType.DMA((NB,)),
    ],
)
def _deg_kernel(rowp, zrs16, deg16, dacc, rowb, srow, onesv, isem, ssem):
    c = lax.axis_index("c")
    s = lax.axis_index("s")
    c_lo = c * H
    ebase = s * PER_SUB
    iota = lax.iota(jnp.int32, 16)
    pltpu.sync_copy(zrs16.at[pl.ds(s * CH, CH)], dacc.at[pl.ds(s * CH, CH)])
    pat = jnp.where(iota == 0, jnp.float32(1.0), jnp.float32(0.0))
    for k in range(K):
        onesv[k, :] = pat
    plsc.subcore_barrier()

    pltpu.async_copy(rowp.at[pl.ds(ebase, BLK)], rowb.at[0], isem.at[0])

    def block(b, nch_prev):
        p = jnp.bitwise_and(b, 1)
        pltpu.make_async_copy(
            rowp.at[pl.ds(ebase + b * BLK, BLK)], rowb.at[p], isem.at[p]
        ).wait()
        @pl.when(b + 1 < NBLK)
        def _():
            pltpu.async_copy(rowp.at[pl.ds(ebase + (b + 1) * BLK, BLK)],
                             rowb.at[1 - p], isem.at[1 - p])
        for j in range(NB):
            @pl.when(j < nch_prev)
            def _():
                pltpu.make_async_copy(
                    onesv, dacc.at[srow.at[1 - p, j]], ssem.at[j]).wait()
        wpos = jnp.int32(0)
        for j in range(BLK // 16):
            r = rowb[p, pl.ds(j * 16, 16)]
            ok = (r >= c_lo) & (r < c_lo + H)
            oki = jnp.where(ok, jnp.int32(1), jnp.int32(0))
            off = wpos + plsc.cumsum(oki) - 1
            d0 = lax.shift_right_logical(off, 6)
            d1 = jnp.bitwise_and(off, 63)
            plsc.store_scatter(srow.at[p], [d0, d1], r - c_lo, mask=ok)
            wpos = wpos + lax.reduce_sum(oki, axes=(0,))
        nch = lax.shift_right_logical(wpos + (K - 1), 6)
        lim = nch * K
        for j in range(4):
            idxs = wpos + j * 16 + iota
            m = idxs < lim
            d0 = lax.shift_right_logical(idxs, 6)
            d1 = jnp.bitwise_and(idxs, 63)
            plsc.store_scatter(srow.at[p], [d0, d1],
                               jnp.full((16,), H, jnp.int32), mask=m)
        for j in range(NB):
            @pl.when(j < nch)
            def _():
                pltpu.async_copy(onesv, dacc.at[srow.at[p, j]],
                                 ssem.at[j], add=True)
        return nch

    nch_last = lax.fori_loop(0, NBLK, block, jnp.int32(0))
    lastp = (NBLK - 1) & 1
    for j in range(NB):
        @pl.when(j < nch_last)
        def _():
            pltpu.make_async_copy(
                onesv, dacc.at[srow.at[lastp, j]], ssem.at[j]).wait()
    plsc.subcore_barrier()
    ob = jnp.minimum(s * CH, H - CH)
    pltpu.sync_copy(dacc.at[pl.ds(ob, CH)], deg16.at[pl.ds(c_lo + ob, CH)])


@functools.partial(
    pl.kernel,
    out_type=_f32((NN, DD)),
    mesh=_mesh,
    compiler_params=pltpu.CompilerParams(use_tc_tiling_on_sc=False,
                                         needs_layout_passes=False),
    scratch_types=[
        pltpu.VMEM_SHARED((HP, DD), jnp.float32),
        pltpu.VMEM((2, NB, K), jnp.int32),
        pltpu.VMEM((2, NB, K), jnp.int32),
        pltpu.VMEM((16,), jnp.int32),
        pltpu.VMEM((NB, K, DD), jnp.float32),
        pltpu.SemaphoreType.DMA((2,)),
        pltpu.SemaphoreType.DMA((NB,)),
        pltpu.SemaphoreType.DMA((NB,)),
    ],
)
def _spmm_kernel(y, scolP, srowP, cnts, zrs, out,
                 acc, colb, rowb, cntv, gbuf, isem, gsem, ssem):
    c = lax.axis_index("c")
    s = lax.axis_index("s")
    c_lo = c * H
    w = c * 16 + s
    obase = w * CAPC
    pltpu.sync_copy(zrs.at[pl.ds(s * CH, CH)], acc.at[pl.ds(s * CH, CH)])
    pltpu.sync_copy(cnts.at[w], cntv)
    nblk = lax.reduce_max(cntv[...], axes=(0,))
    plsc.subcore_barrier()

    pltpu.async_copy(scolP.at[pl.ds(obase, NB)], colb.at[0], isem.at[0])
    pltpu.async_copy(srowP.at[pl.ds(obase, NB)], rowb.at[0], isem.at[0])

    def block(b, carry):
        p = jnp.bitwise_and(b, 1)
        base = obase + b * NB
        pltpu.make_async_copy(
            scolP.at[pl.ds(base, NB)], colb.at[p], isem.at[p]).wait()
        pltpu.make_async_copy(
            srowP.at[pl.ds(base, NB)], rowb.at[p], isem.at[p]).wait()

        @pl.when(b + 1 < nblk)
        def _():
            nbase = base + NB
            pltpu.async_copy(scolP.at[pl.ds(nbase, NB)],
                             colb.at[1 - p], isem.at[1 - p])
            pltpu.async_copy(srowP.at[pl.ds(nbase, NB)],
                             rowb.at[1 - p], isem.at[1 - p])
        @pl.when(b > 0)
        def _():
            for j in range(NB):
                pltpu.make_async_copy(
                    gbuf.at[j], acc.at[rowb.at[1 - p, j]], ssem.at[j]).wait()
        gd = [pltpu.async_copy(y.at[colb.at[p, j]],
                               gbuf.at[j], gsem.at[j]) for j in range(NB)]
        for j in range(NB):
            gd[j].wait()
            pltpu.async_copy(gbuf.at[j], acc.at[rowb.at[p, j]],
                             ssem.at[j], add=True)
        return carry

    lax.fori_loop(0, nblk, block, 0)
    lastp = jnp.bitwise_and(nblk - 1, 1)
    for j in range(NB):
        pltpu.make_async_copy(
            gbuf.at[j], acc.at[rowb.at[lastp, j]], ssem.at[j]).wait()
    plsc.subcore_barrier()
    ob = jnp.minimum(s * CH, H - CH)
    pltpu.sync_copy(acc.at[pl.ds(ob, CH)], out.at[pl.ds(c_lo + ob, CH)])


@functools.partial(
    pl.kernel,
    out_type=(_f32((BB, DD)), _f32((BB, DD)), _f32((BB, DD))),
    mesh=_mesh,
    compiler_params=pltpu.CompilerParams(use_tc_tiling_on_sc=False, needs_layout_passes=False),
    scratch_types=[
        pltpu.VMEM((128,), jnp.int32),
        pltpu.VMEM((128, DD), jnp.float32),
        pltpu.SemaphoreType.DMA,
    ],
)
def _bpr_gather_kernel(S, users, items, negs, anc, pos, neg, idxv, buf, sem):
    c = lax.axis_index("c")
    s = lax.axis_index("s")
    base = (s * 2 + c) * 128

    pltpu.sync_copy(users.at[pl.ds(base, 128)], idxv)
    pltpu.async_copy(S.at[idxv], buf, sem).wait()
    pltpu.sync_copy(buf, anc.at[pl.ds(base, 128)])

    for src, dst in ((items, pos), (negs, neg)):
        pltpu.sync_copy(src.at[pl.ds(base, 128)], idxv)
        for j in range(8):
            idxv[pl.ds(j * 16, 16)] = idxv[pl.ds(j * 16, 16)] + N_U
        pltpu.async_copy(S.at[idxv], buf, sem).wait()
        pltpu.sync_copy(buf, dst.at[pl.ds(base, 128)])


_RB = 2000


def _prep_body(deg_ref, x0_ref, dis_ref, y0_ref):
    d = jnp.clip(lax.rsqrt(deg_ref[:, 0:1] + 1e-6), 0.0, 10.0)
    dis_ref[...] = d
    y0_ref[...] = d * x0_ref[...]


_prep = pl.pallas_call(
    _prep_body,
    grid=(NN // _RB,),
    in_specs=[
        pl.BlockSpec((_RB, 16), lambda i: (i, 0)),
        pl.BlockSpec((_RB, DD), lambda i: (i, 0)),
    ],
    out_specs=[
        pl.BlockSpec((_RB, 1), lambda i: (i, 0)),
        pl.BlockSpec((_RB, DD), lambda i: (i, 0)),
    ],
    out_shape=[_f32((NN, 1)), _f32((NN, DD))],
)


def _scale_body(acc_ref, dis_ref, s_ref, snew_ref, y_ref):
    d = dis_ref[...]
    da = d * acc_ref[...]
    snew_ref[...] = s_ref[...] + da
    y_ref[...] = d * da


_scale = pl.pallas_call(
    _scale_body,
    grid=(NN // _RB,),
    in_specs=[
        pl.BlockSpec((_RB, DD), lambda i: (i, 0)),
        pl.BlockSpec((_RB, 1), lambda i: (i, 0)),
        pl.BlockSpec((_RB, DD), lambda i: (i, 0)),
    ],
    out_specs=[
        pl.BlockSpec((_RB, DD), lambda i: (i, 0)),
        pl.BlockSpec((_RB, DD), lambda i: (i, 0)),
    ],
    out_shape=[_f32((NN, DD)), _f32((NN, DD))],
)


def _loss_body(a_ref, p_ref, n_ref, o_ref):
    a = a_ref[...]
    p = p_ref[...]
    n = n_ref[...]
    diff = jnp.sum(a * p, axis=-1) - jnp.sum(a * n, axis=-1)
    bpr = -jnp.sum(jnp.log(jax.nn.sigmoid(diff) + 1e-12)) / float(BB)
    reg = 0.5 * (jnp.sum(a * a) + jnp.sum(p * p) + jnp.sum(n * n)) / float(BB)
    o_ref[...] = (bpr + reg).reshape(1, 1)


_loss = pl.pallas_call(
    _loss_body,
    out_shape=_f32((1, 1)),
)


def kernel(user_emb, item_emb, edge_index, users, items, neg_items):
    x0 = jnp.concatenate([user_emb, item_emb], axis=0)
    pad = EP - EE
    colp = jnp.concatenate([edge_index[1], jnp.zeros((pad,), jnp.int32)])
    rowp = jnp.concatenate([edge_index[0], jnp.full((pad,), -1, jnp.int32)])
    zrs = jnp.zeros((HP, DD), jnp.float32)
    zrs16 = jnp.zeros((HP, 16), jnp.float32)

    srowP, scolP, cnts = _part_kernel(colp, rowp)
    deg16 = _deg_kernel(rowp, zrs16)
    dis, y = _prep(deg16, x0)
    s_sum = x0
    for _ in range(3):
        acc = _spmm_kernel(y, scolP, srowP, cnts, zrs)
        s_sum, y = _scale(acc, dis, s_sum)
    anc, pos, neg = _bpr_gather_kernel(s_sum, users, items, neg_items)
    return _loss(anc, pos, neg)[0, 0]


# speedup vs baseline: 5.9149x; 1.0226x over previous
import functools

import jax
import jax.numpy as jnp
from jax import lax
from jax.experimental import pallas as pl
from jax.experimental.pallas import tpu as pltpu
from jax.experimental.pallas import tpu_sc as plsc

N_U = 30000
N_I = 20000
NN = 50000
EE = 800000
DD = 64
BB = 4096

H = 25000
CH = 1568
HP = 16 * CH
K = 64
NB = 6
BLK = NB * K
NBLK = 131
PER_SUB = NBLK * BLK
EP = 16 * PER_SUB

CAP = 50688
CAPC = CAP // K
NCHT = 32 * CAPC
NSTG = 8

_mesh = plsc.VectorSubcoreMesh(
    core_axis_name="c", subcore_axis_name="s", num_cores=2, num_subcores=16)


def _f32(shape):
    return jax.ShapeDtypeStruct(shape, jnp.float32)


def _i32(shape):
    return jax.ShapeDtypeStruct(shape, jnp.int32)


@functools.partial(
    pl.kernel,
    out_type=(_i32((NCHT, K)), _i32((NCHT, K)), _i32((32, 16)),
              _f32((NN, 16))),
    mesh=_mesh,
    compiler_params=pltpu.CompilerParams(use_tc_tiling_on_sc=False,
                                         needs_layout_passes=False),
    scratch_types=[
        pltpu.VMEM((2, BLK), jnp.int32),
        pltpu.VMEM((2, BLK), jnp.int32),
        pltpu.VMEM((2, NSTG, K), jnp.int32),
        pltpu.VMEM((2, NSTG, K), jnp.int32),
        pltpu.VMEM((16,), jnp.int32),
        pltpu.VMEM_SHARED((HP, 16), jnp.float32),
        pltpu.VMEM((K, 16), jnp.float32),
        pltpu.SemaphoreType.DMA((2,)),
        pltpu.SemaphoreType.DMA((NSTG,)),
        pltpu.SemaphoreType.DMA((NSTG,)),
    ],
)
def _part_kernel(colp, rowp, zrs16, srowP, scolP, cnts, deg16, colb, rowb,
                 sstg, cstg, cntv, dacc, onesv, isem, fsem, dsem):
    c = lax.axis_index("c")
    s = lax.axis_index("s")
    w = c * 16 + s
    obase = w * CAPC
    c_lo = c * H
    ebase = s * PER_SUB
    iota = lax.iota(jnp.int32, 16)

    pltpu.async_copy(colp.at[pl.ds(ebase, BLK)], colb.at[0], isem.at[0])
    pltpu.async_copy(rowp.at[pl.ds(ebase, BLK)], rowb.at[0], isem.at[0])
    pltpu.sync_copy(zrs16.at[pl.ds(s * CH, CH)], dacc.at[pl.ds(s * CH, CH)])
    pat = jnp.where(iota == 0, jnp.float32(1.0), jnp.float32(0.0))
    for k in range(K):
        onesv[k, :] = pat
    plsc.subcore_barrier()

    def block(b, carry):
        wpos, ocnt, nfl_prev = carry
        p = jnp.bitwise_and(b, 1)
        base = ebase + b * BLK
        pltpu.make_async_copy(
            colp.at[pl.ds(base, BLK)], colb.at[p], isem.at[p]).wait()
        pltpu.make_async_copy(
            rowp.at[pl.ds(base, BLK)], rowb.at[p], isem.at[p]).wait()

        @pl.when(b + 1 < NBLK)
        def _():
            nbase = base + BLK
            pltpu.async_copy(colp.at[pl.ds(nbase, BLK)],
                             colb.at[1 - p], isem.at[1 - p])
            pltpu.async_copy(rowp.at[pl.ds(nbase, BLK)],
                             rowb.at[1 - p], isem.at[1 - p])
        pch = obase + ocnt - nfl_prev
        for j in range(NSTG - 1):
            @pl.when(j < nfl_prev)
            def _():
                pltpu.make_async_copy(
                    sstg.at[1 - p, j], srowP.at[pch + j], fsem.at[j]).wait()
                pltpu.make_async_copy(
                    cstg.at[1 - p, j], scolP.at[pch + j], fsem.at[j]).wait()
                pltpu.make_async_copy(
                    onesv, dacc.at[sstg.at[1 - p, j]], dsem.at[j]).wait()
        for j in range(BLK // 16):
            r = rowb[p, pl.ds(j * 16, 16)]
            cv = colb[p, pl.ds(j * 16, 16)]
            ok = (r >= c_lo) & (r < c_lo + H)
            oki = jnp.where(ok, jnp.int32(1), jnp.int32(0))
            off = wpos + plsc.cumsum(oki) - 1
            d0 = lax.shift_right_logical(off, 6)
            d1 = jnp.bitwise_and(off, 63)
            plsc.store_scatter(sstg.at[p], [d0, d1], r - c_lo, mask=ok)
            plsc.store_scatter(cstg.at[p], [d0, d1], cv, mask=ok)
            wpos = wpos + lax.reduce_sum(oki, axes=(0,))
        nfl = lax.shift_right_logical(wpos, 6)
        fch = obase + ocnt
        for j in range(NSTG - 1):
            @pl.when(j < nfl)
            def _():
                pltpu.async_copy(sstg.at[p, j], srowP.at[fch + j], fsem.at[j])
                pltpu.async_copy(cstg.at[p, j], scolP.at[fch + j], fsem.at[j])
                pltpu.async_copy(onesv, dacc.at[sstg.at[p, j]],
                                 dsem.at[j], add=True)
        for i in range(4):
            sl = pl.ds(i * 16, 16)
            sstg[1 - p, 0, sl] = sstg[p, nfl, sl]
            cstg[1 - p, 0, sl] = cstg[p, nfl, sl]
        return jnp.bitwise_and(wpos, 63), ocnt + nfl, nfl

    wpos, ocnt, nfl_last = lax.fori_loop(
        0, NBLK, block, (jnp.int32(0), jnp.int32(0), jnp.int32(0)))
    lastp = (NBLK - 1) & 1
    pch = obase + ocnt - nfl_last
    for j in range(NSTG - 1):
        @pl.when(j < nfl_last)
        def _():
            pltpu.make_async_copy(
                sstg.at[1 - lastp, j], srowP.at[pch + j], fsem.at[j]).wait()
            pltpu.make_async_copy(
                cstg.at[1 - lastp, j], scolP.at[pch + j], fsem.at[j]).wait()
            pltpu.make_async_copy(
                onesv, dacc.at[sstg.at[1 - lastp, j]], dsem.at[j]).wait()
    fp = 1 - lastp
    for i in range(4):
        idxs = i * 16 + iota
        m = idxs >= wpos
        sl = pl.ds(i * 16, 16)
        plsc.store_scatter(sstg.at[fp, 0], [idxs], jnp.full((16,), H,
                                                           jnp.int32), mask=m)
        plsc.store_scatter(cstg.at[fp, 0], [idxs], jnp.zeros((16,),
                                                            jnp.int32), mask=m)
        sstg[fp, 1, sl] = jnp.full((16,), H, jnp.int32)
        cstg[fp, 1, sl] = jnp.zeros((16,), jnp.int32)
    pltpu.sync_copy(sstg.at[fp, 0], srowP.at[obase + ocnt])
    pltpu.sync_copy(cstg.at[fp, 0], scolP.at[obase + ocnt])
    pltpu.sync_copy(onesv, dacc.at[sstg.at[fp, 0]], add=True)
    ocnt = ocnt + 1
    nblk6 = lax.shift_right_logical(ocnt * 43691, 18)
    npad = nblk6 * 6 + 6 - ocnt
    npad = jnp.where(npad == 6, 0, npad)
    for j in range(5):
        @pl.when(j < npad)
        def _():
            pb = obase + ocnt + j
            pltpu.sync_copy(sstg.at[fp, 1], srowP.at[pb])
            pltpu.sync_copy(cstg.at[fp, 1], scolP.at[pb])
    ocnt = ocnt + npad
    nblk = lax.shift_right_logical(ocnt * 43691, 18)
    cntv[...] = jnp.full((16,), 1, jnp.int32) * nblk
    pltpu.sync_copy(cntv, cnts.at[w])
    plsc.subcore_barrier()
    ob = jnp.minimum(s * CH, H - CH)
    pltpu.sync_copy(dacc.at[pl.ds(ob, CH)], deg16.at[pl.ds(c_lo + ob, CH)])


@functools.partial(
    pl.kernel,
    out_type=_f32((NN, DD)),
    mesh=_mesh,
    compiler_params=pltpu.CompilerParams(use_tc_tiling_on_sc=False,
                                         needs_layout_passes=False),
    scratch_types=[
        pltpu.VMEM_SHARED((HP, DD), jnp.float32),
        pltpu.VMEM((2, NB, K), jnp.int32),
        pltpu.VMEM((2, NB, K), jnp.int32),
        pltpu.VMEM((16,), jnp.int32),
        pltpu.VMEM((NB, K, DD), jnp.float32),
        pltpu.SemaphoreType.DMA((2,)),
        pltpu.SemaphoreType.DMA((NB,)),
        pltpu.SemaphoreType.DMA((NB,)),
    ],
)
def _spmm_kernel(y, scolP, srowP, cnts, zrs, out,
                 acc, colb, rowb, cntv, gbuf, isem, gsem, ssem):
    c = lax.axis_index("c")
    s = lax.axis_index("s")
    c_lo = c * H
    w = c * 16 + s
    obase = w * CAPC
    pltpu.sync_copy(zrs.at[pl.ds(s * CH, CH)], acc.at[pl.ds(s * CH, CH)])
    pltpu.sync_copy(cnts.at[w], cntv)
    nblk = lax.reduce_max(cntv[...], axes=(0,))
    plsc.subcore_barrier()

    pltpu.async_copy(scolP.at[pl.ds(obase, NB)], colb.at[0], isem.at[0])
    pltpu.async_copy(srowP.at[pl.ds(obase, NB)], rowb.at[0], isem.at[0])

    def block(b, carry):
        p = jnp.bitwise_and(b, 1)
        base = obase + b * NB
        pltpu.make_async_copy(
            scolP.at[pl.ds(base, NB)], colb.at[p], isem.at[p]).wait()
        pltpu.make_async_copy(
            srowP.at[pl.ds(base, NB)], rowb.at[p], isem.at[p]).wait()

        @pl.when(b + 1 < nblk)
        def _():
            nbase = base + NB
            pltpu.async_copy(scolP.at[pl.ds(nbase, NB)],
                             colb.at[1 - p], isem.at[1 - p])
            pltpu.async_copy(srowP.at[pl.ds(nbase, NB)],
                             rowb.at[1 - p], isem.at[1 - p])
        @pl.when(b > 0)
        def _():
            for j in range(NB):
                pltpu.make_async_copy(
                    gbuf.at[j], acc.at[rowb.at[1 - p, j]], ssem.at[j]).wait()
        gd = [pltpu.async_copy(y.at[colb.at[p, j]],
                               gbuf.at[j], gsem.at[j]) for j in range(NB)]
        for j in range(NB):
            gd[j].wait()
            pltpu.async_copy(gbuf.at[j], acc.at[rowb.at[p, j]],
                             ssem.at[j], add=True)
        return carry

    lax.fori_loop(0, nblk, block, 0)
    lastp = jnp.bitwise_and(nblk - 1, 1)
    for j in range(NB):
        pltpu.make_async_copy(
            gbuf.at[j], acc.at[rowb.at[lastp, j]], ssem.at[j]).wait()
    plsc.subcore_barrier()
    ob = jnp.minimum(s * CH, H - CH)
    pltpu.sync_copy(acc.at[pl.ds(ob, CH)], out.at[pl.ds(c_lo + ob, CH)])


@functools.partial(
    pl.kernel,
    out_type=(_f32((BB, DD)), _f32((BB, DD)), _f32((BB, DD))),
    mesh=_mesh,
    compiler_params=pltpu.CompilerParams(use_tc_tiling_on_sc=False, needs_layout_passes=False),
    scratch_types=[
        pltpu.VMEM((128,), jnp.int32),
        pltpu.VMEM((128, DD), jnp.float32),
        pltpu.SemaphoreType.DMA,
    ],
)
def _bpr_gather_kernel(S, users, items, negs, anc, pos, neg, idxv, buf, sem):
    c = lax.axis_index("c")
    s = lax.axis_index("s")
    base = (s * 2 + c) * 128

    pltpu.sync_copy(users.at[pl.ds(base, 128)], idxv)
    pltpu.async_copy(S.at[idxv], buf, sem).wait()
    pltpu.sync_copy(buf, anc.at[pl.ds(base, 128)])

    for src, dst in ((items, pos), (negs, neg)):
        pltpu.sync_copy(src.at[pl.ds(base, 128)], idxv)
        for j in range(8):
            idxv[pl.ds(j * 16, 16)] = idxv[pl.ds(j * 16, 16)] + N_U
        pltpu.async_copy(S.at[idxv], buf, sem).wait()
        pltpu.sync_copy(buf, dst.at[pl.ds(base, 128)])


_RB = 2000


def _prep_body(deg_ref, x0_ref, dis_ref, y0_ref):
    d = jnp.clip(lax.rsqrt(deg_ref[:, 0:1] + 1e-6), 0.0, 10.0)
    dis_ref[...] = d
    y0_ref[...] = d * x0_ref[...]


_prep = pl.pallas_call(
    _prep_body,
    grid=(NN // _RB,),
    in_specs=[
        pl.BlockSpec((_RB, 16), lambda i: (i, 0)),
        pl.BlockSpec((_RB, DD), lambda i: (i, 0)),
    ],
    out_specs=[
        pl.BlockSpec((_RB, 1), lambda i: (i, 0)),
        pl.BlockSpec((_RB, DD), lambda i: (i, 0)),
    ],
    out_shape=[_f32((NN, 1)), _f32((NN, DD))],
)


def _scale_body(acc_ref, dis_ref, s_ref, snew_ref, y_ref):
    d = dis_ref[...]
    da = d * acc_ref[...]
    snew_ref[...] = s_ref[...] + da
    y_ref[...] = d * da


_scale = pl.pallas_call(
    _scale_body,
    grid=(NN // _RB,),
    in_specs=[
        pl.BlockSpec((_RB, DD), lambda i: (i, 0)),
        pl.BlockSpec((_RB, 1), lambda i: (i, 0)),
        pl.BlockSpec((_RB, DD), lambda i: (i, 0)),
    ],
    out_specs=[
        pl.BlockSpec((_RB, DD), lambda i: (i, 0)),
        pl.BlockSpec((_RB, DD), lambda i: (i, 0)),
    ],
    out_shape=[_f32((NN, DD)), _f32((NN, DD))],
)


def _loss_body(a_ref, p_ref, n_ref, o_ref):
    a = a_ref[...]
    p = p_ref[...]
    n = n_ref[...]
    diff = jnp.sum(a * p, axis=-1) - jnp.sum(a * n, axis=-1)
    bpr = -jnp.sum(jnp.log(jax.nn.sigmoid(diff) + 1e-12)) / float(BB)
    reg = 0.5 * (jnp.sum(a * a) + jnp.sum(p * p) + jnp.sum(n * n)) / float(BB)
    o_ref[...] = (bpr + reg).reshape(1, 1)


_loss = pl.pallas_call(
    _loss_body,
    out_shape=_f32((1, 1)),
)


def kernel(user_emb, item_emb, edge_index, users, items, neg_items):
    x0 = jnp.concatenate([user_emb, item_emb], axis=0)
    pad = EP - EE
    colp = jnp.concatenate([edge_index[1], jnp.zeros((pad,), jnp.int32)])
    rowp = jnp.concatenate([edge_index[0], jnp.full((pad,), -1, jnp.int32)])
    zrs = jnp.zeros((HP, DD), jnp.float32)
    zrs16 = jnp.zeros((HP, 16), jnp.float32)

    srowP, scolP, cnts, deg16 = _part_kernel(colp, rowp, zrs16)
    dis, y = _prep(deg16, x0)
    s_sum = x0
    for _ in range(3):
        acc = _spmm_kernel(y, scolP, srowP, cnts, zrs)
        s_sum, y = _scale(acc, dis, s_sum)
    anc, pos, neg = _bpr_gather_kernel(s_sum, users, items, neg_items)
    return _loss(anc, pos, neg)[0, 0]


# speedup vs baseline: 6.1627x; 1.0419x over previous
import functools

import jax
import jax.numpy as jnp
from jax import lax
from jax.experimental import pallas as pl
from jax.experimental.pallas import tpu as pltpu
from jax.experimental.pallas import tpu_sc as plsc

N_U = 30000
N_I = 20000
NN = 50000
EE = 800000
DD = 64
BB = 4096

H = 25000
CH = 1568
HP = 16 * CH
K = 64
NB = 6
BLK = NB * K
NBLK = 131
PER_SUB = NBLK * BLK
EP = 16 * PER_SUB

CAP = 50688
CAPC = CAP // K
NCHT = 32 * CAPC
NSTG = 8

_mesh = plsc.VectorSubcoreMesh(
    core_axis_name="c", subcore_axis_name="s", num_cores=2, num_subcores=16)


def _f32(shape):
    return jax.ShapeDtypeStruct(shape, jnp.float32)


def _i32(shape):
    return jax.ShapeDtypeStruct(shape, jnp.int32)


@functools.partial(
    pl.kernel,
    out_type=(_i32((NCHT, K)), _i32((NCHT, K)), _i32((32, 16)),
              _f32((NN, 16))),
    mesh=_mesh,
    compiler_params=pltpu.CompilerParams(use_tc_tiling_on_sc=False,
                                         needs_layout_passes=False),
    scratch_types=[
        pltpu.VMEM((2, BLK), jnp.int32),
        pltpu.VMEM((2, BLK), jnp.int32),
        pltpu.VMEM((2, NSTG, K), jnp.int32),
        pltpu.VMEM((2, NSTG, K), jnp.int32),
        pltpu.VMEM((16,), jnp.int32),
        pltpu.VMEM_SHARED((HP, 16), jnp.float32),
        pltpu.VMEM((K, 16), jnp.float32),
        pltpu.SemaphoreType.DMA((2,)),
        pltpu.SemaphoreType.DMA((NSTG,)),
        pltpu.SemaphoreType.DMA((NSTG,)),
    ],
)
def _part_kernel(colp, rowp, zrs16, srowP, scolP, cnts, deg16, colb, rowb,
                 sstg, cstg, cntv, dacc, onesv, isem, fsem, dsem):
    c = lax.axis_index("c")
    s = lax.axis_index("s")
    w = c * 16 + s
    obase = w * CAPC
    c_lo = c * H
    ebase = s * PER_SUB
    iota = lax.iota(jnp.int32, 16)

    pltpu.async_copy(colp.at[pl.ds(ebase, BLK)], colb.at[0], isem.at[0])
    pltpu.async_copy(rowp.at[pl.ds(ebase, BLK)], rowb.at[0], isem.at[0])
    pltpu.sync_copy(zrs16.at[pl.ds(s * CH, CH)], dacc.at[pl.ds(s * CH, CH)])
    pat = jnp.where(iota == 0, jnp.float32(1.0), jnp.float32(0.0))
    for k in range(K):
        onesv[k, :] = pat
    plsc.subcore_barrier()

    def block(b, carry):
        wpos, ocnt, nfl_prev = carry
        p = jnp.bitwise_and(b, 1)
        base = ebase + b * BLK
        pltpu.make_async_copy(
            colp.at[pl.ds(base, BLK)], colb.at[p], isem.at[p]).wait()
        pltpu.make_async_copy(
            rowp.at[pl.ds(base, BLK)], rowb.at[p], isem.at[p]).wait()

        @pl.when(b + 1 < NBLK)
        def _():
            nbase = base + BLK
            pltpu.async_copy(colp.at[pl.ds(nbase, BLK)],
                             colb.at[1 - p], isem.at[1 - p])
            pltpu.async_copy(rowp.at[pl.ds(nbase, BLK)],
                             rowb.at[1 - p], isem.at[1 - p])
        pch = obase + ocnt - nfl_prev
        for j in range(NSTG - 1):
            @pl.when(j < nfl_prev)
            def _():
                pltpu.make_async_copy(
                    sstg.at[1 - p, j], srowP.at[pch + j], fsem.at[j]).wait()
                pltpu.make_async_copy(
                    cstg.at[1 - p, j], scolP.at[pch + j], fsem.at[j]).wait()
                pltpu.make_async_copy(
                    onesv, dacc.at[sstg.at[1 - p, j]], dsem.at[j]).wait()
        for j in range(BLK // 16):
            r = rowb[p, pl.ds(j * 16, 16)]
            cv = colb[p, pl.ds(j * 16, 16)]
            ok = (r >= c_lo) & (r < c_lo + H)
            oki = jnp.where(ok, jnp.int32(1), jnp.int32(0))
            off = wpos + plsc.cumsum(oki) - 1
            d0 = lax.shift_right_logical(off, 6)
            d1 = jnp.bitwise_and(off, 63)
            plsc.store_scatter(sstg.at[p], [d0, d1], r - c_lo, mask=ok)
            plsc.store_scatter(cstg.at[p], [d0, d1], cv, mask=ok)
            wpos = wpos + lax.reduce_sum(oki, axes=(0,))
        nfl = lax.shift_right_logical(wpos, 6)
        fch = obase + ocnt
        for j in range(NSTG - 1):
            @pl.when(j < nfl)
            def _():
                pltpu.async_copy(sstg.at[p, j], srowP.at[fch + j], fsem.at[j])
                pltpu.async_copy(cstg.at[p, j], scolP.at[fch + j], fsem.at[j])
                pltpu.async_copy(onesv, dacc.at[sstg.at[p, j]],
                                 dsem.at[j], add=True)
        for i in range(4):
            sl = pl.ds(i * 16, 16)
            sstg[1 - p, 0, sl] = sstg[p, nfl, sl]
            cstg[1 - p, 0, sl] = cstg[p, nfl, sl]
        return jnp.bitwise_and(wpos, 63), ocnt + nfl, nfl

    wpos, ocnt, nfl_last = lax.fori_loop(
        0, NBLK, block, (jnp.int32(0), jnp.int32(0), jnp.int32(0)))
    lastp = (NBLK - 1) & 1
    pch = obase + ocnt - nfl_last
    for j in range(NSTG - 1):
        @pl.when(j < nfl_last)
        def _():
            pltpu.make_async_copy(
                sstg.at[1 - lastp, j], srowP.at[pch + j], fsem.at[j]).wait()
            pltpu.make_async_copy(
                cstg.at[1 - lastp, j], scolP.at[pch + j], fsem.at[j]).wait()
            pltpu.make_async_copy(
                onesv, dacc.at[sstg.at[1 - lastp, j]], dsem.at[j]).wait()
    fp = 1 - lastp
    for i in range(4):
        idxs = i * 16 + iota
        m = idxs >= wpos
        sl = pl.ds(i * 16, 16)
        plsc.store_scatter(sstg.at[fp, 0], [idxs], jnp.full((16,), H,
                                                           jnp.int32), mask=m)
        plsc.store_scatter(cstg.at[fp, 0], [idxs], jnp.zeros((16,),
                                                            jnp.int32), mask=m)
        sstg[fp, 1, sl] = jnp.full((16,), H, jnp.int32)
        cstg[fp, 1, sl] = jnp.zeros((16,), jnp.int32)
    pltpu.sync_copy(sstg.at[fp, 0], srowP.at[obase + ocnt])
    pltpu.sync_copy(cstg.at[fp, 0], scolP.at[obase + ocnt])
    pltpu.sync_copy(onesv, dacc.at[sstg.at[fp, 0]], add=True)
    ocnt = ocnt + 1
    nblk6 = lax.shift_right_logical(ocnt * 43691, 18)
    npad = nblk6 * 6 + 6 - ocnt
    npad = jnp.where(npad == 6, 0, npad)
    for j in range(5):
        @pl.when(j < npad)
        def _():
            pb = obase + ocnt + j
            pltpu.sync_copy(sstg.at[fp, 1], srowP.at[pb])
            pltpu.sync_copy(cstg.at[fp, 1], scolP.at[pb])
    ocnt = ocnt + npad
    nblk = lax.shift_right_logical(ocnt * 43691, 18)
    cntv[...] = jnp.full((16,), 1, jnp.int32) * nblk
    pltpu.sync_copy(cntv, cnts.at[w])
    plsc.subcore_barrier()
    ob = jnp.minimum(s * CH, H - CH)
    pltpu.sync_copy(dacc.at[pl.ds(ob, CH)], deg16.at[pl.ds(c_lo + ob, CH)])


@functools.partial(
    pl.kernel,
    out_type=(_f32((NN, DD)), _f32((NN, DD))),
    mesh=_mesh,
    compiler_params=pltpu.CompilerParams(use_tc_tiling_on_sc=False,
                                         needs_layout_passes=False),
    scratch_types=[
        pltpu.VMEM_SHARED((HP, DD), jnp.float32),
        pltpu.VMEM((2, NB, K), jnp.int32),
        pltpu.VMEM((2, NB, K), jnp.int32),
        pltpu.VMEM((16,), jnp.int32),
        pltpu.VMEM((BLK, DD), jnp.float32),
        pltpu.VMEM((224, 16), jnp.float32),
        pltpu.SemaphoreType.DMA((2,)),
        pltpu.SemaphoreType.DMA((NB,)),
        pltpu.SemaphoreType.DMA((NB,)),
    ],
)
def _spmm_kernel(y, scolP, srowP, cnts, zrs, dis16, xl, ynext,
                 acc, colb, rowb, cntv, gbuf, dbuf, isem, gsem, ssem):
    c = lax.axis_index("c")
    s = lax.axis_index("s")
    c_lo = c * H
    w = c * 16 + s
    obase = w * CAPC
    pltpu.sync_copy(zrs.at[pl.ds(s * CH, CH)], acc.at[pl.ds(s * CH, CH)])
    pltpu.sync_copy(cnts.at[w], cntv)
    nblk = lax.reduce_max(cntv[...], axes=(0,))
    plsc.subcore_barrier()

    pltpu.async_copy(scolP.at[pl.ds(obase, NB)], colb.at[0], isem.at[0])
    pltpu.async_copy(srowP.at[pl.ds(obase, NB)], rowb.at[0], isem.at[0])

    def block(b, carry):
        p = jnp.bitwise_and(b, 1)
        base = obase + b * NB
        pltpu.make_async_copy(
            scolP.at[pl.ds(base, NB)], colb.at[p], isem.at[p]).wait()
        pltpu.make_async_copy(
            srowP.at[pl.ds(base, NB)], rowb.at[p], isem.at[p]).wait()

        @pl.when(b + 1 < nblk)
        def _():
            nbase = base + NB
            pltpu.async_copy(scolP.at[pl.ds(nbase, NB)],
                             colb.at[1 - p], isem.at[1 - p])
            pltpu.async_copy(srowP.at[pl.ds(nbase, NB)],
                             rowb.at[1 - p], isem.at[1 - p])
        @pl.when(b > 0)
        def _():
            for j in range(NB):
                pltpu.make_async_copy(
                    gbuf.at[pl.ds(j * K, K)], acc.at[rowb.at[1 - p, j]],
                    ssem.at[j]).wait()
        gd = [pltpu.async_copy(y.at[colb.at[p, j]],
                               gbuf.at[pl.ds(j * K, K)], gsem.at[j])
              for j in range(NB)]
        for j in range(NB):
            gd[j].wait()
            pltpu.async_copy(gbuf.at[pl.ds(j * K, K)], acc.at[rowb.at[p, j]],
                             ssem.at[j], add=True)
        return carry

    lax.fori_loop(0, nblk, block, 0)
    lastp = jnp.bitwise_and(nblk - 1, 1)
    for j in range(NB):
        pltpu.make_async_copy(
            gbuf.at[pl.ds(j * K, K)], acc.at[rowb.at[lastp, j]],
            ssem.at[j]).wait()
    plsc.subcore_barrier()
    ob = jnp.minimum(s * CH, H - CH)

    def subblk(t, carry):
        r0 = ob + t * 224
        pltpu.sync_copy(acc.at[pl.ds(r0, 224)], gbuf.at[pl.ds(0, 224)])
        pltpu.sync_copy(dis16.at[pl.ds(c_lo + r0, 224)], dbuf)

        def rowscale(rr, carry2):
            d = dbuf[rr, :]
            for q in range(DD // 16):
                sl = pl.ds(q * 16, 16)
                gbuf[rr, sl] = gbuf[rr, sl] * d
            return carry2

        lax.fori_loop(0, 224, rowscale, 0)
        pltpu.sync_copy(gbuf.at[pl.ds(0, 224)], xl.at[pl.ds(c_lo + r0, 224)])
        lax.fori_loop(0, 224, rowscale, 0)
        pltpu.sync_copy(gbuf.at[pl.ds(0, 224)],
                        ynext.at[pl.ds(c_lo + r0, 224)])
        return carry

    lax.fori_loop(0, CH // 224, subblk, 0)


@functools.partial(
    pl.kernel,
    out_type=(_f32((BB, DD)), _f32((BB, DD)), _f32((BB, DD))),
    mesh=_mesh,
    compiler_params=pltpu.CompilerParams(use_tc_tiling_on_sc=False, needs_layout_passes=False),
    scratch_types=[
        pltpu.VMEM((128,), jnp.int32),
        pltpu.VMEM((128, DD), jnp.float32),
        pltpu.SemaphoreType.DMA,
    ],
)
def _bpr_gather_kernel(S, users, items, negs, anc, pos, neg, idxv, buf, sem):
    c = lax.axis_index("c")
    s = lax.axis_index("s")
    base = (s * 2 + c) * 128

    pltpu.sync_copy(users.at[pl.ds(base, 128)], idxv)
    pltpu.async_copy(S.at[idxv], buf, sem).wait()
    pltpu.sync_copy(buf, anc.at[pl.ds(base, 128)])

    for src, dst in ((items, pos), (negs, neg)):
        pltpu.sync_copy(src.at[pl.ds(base, 128)], idxv)
        for j in range(8):
            idxv[pl.ds(j * 16, 16)] = idxv[pl.ds(j * 16, 16)] + N_U
        pltpu.async_copy(S.at[idxv], buf, sem).wait()
        pltpu.sync_copy(buf, dst.at[pl.ds(base, 128)])


_RB = 2000


def _prep_body(deg_ref, x0_ref, dis_ref, y0_ref):
    d = jnp.clip(lax.rsqrt(deg_ref[:, 0:1] + 1e-6), 0.0, 10.0)
    dis_ref[...] = jnp.broadcast_to(d, (_RB, 16))
    y0_ref[...] = d * x0_ref[...]


_prep = pl.pallas_call(
    _prep_body,
    grid=(NN // _RB,),
    in_specs=[
        pl.BlockSpec((_RB, 16), lambda i: (i, 0)),
        pl.BlockSpec((_RB, DD), lambda i: (i, 0)),
    ],
    out_specs=[
        pl.BlockSpec((_RB, 16), lambda i: (i, 0)),
        pl.BlockSpec((_RB, DD), lambda i: (i, 0)),
    ],
    out_shape=[_f32((NN, 16)), _f32((NN, DD))],
)


def _loss_body(a_ref, p_ref, n_ref, o_ref):
    a = a_ref[...]
    p = p_ref[...]
    n = n_ref[...]
    diff = jnp.sum(a * p, axis=-1) - jnp.sum(a * n, axis=-1)
    bpr = -jnp.sum(jnp.log(jax.nn.sigmoid(diff) + 1e-12)) / float(BB)
    reg = 0.5 * (jnp.sum(a * a) + jnp.sum(p * p) + jnp.sum(n * n)) / float(BB)
    o_ref[...] = (bpr + reg).reshape(1, 1)


_loss = pl.pallas_call(
    _loss_body,
    out_shape=_f32((1, 1)),
)


def kernel(user_emb, item_emb, edge_index, users, items, neg_items):
    x0 = jnp.concatenate([user_emb, item_emb], axis=0)
    pad = EP - EE
    colp = jnp.concatenate([edge_index[1], jnp.zeros((pad,), jnp.int32)])
    rowp = jnp.concatenate([edge_index[0], jnp.full((pad,), -1, jnp.int32)])
    zrs = jnp.zeros((HP, DD), jnp.float32)
    zrs16 = jnp.zeros((HP, 16), jnp.float32)

    srowP, scolP, cnts, deg16 = _part_kernel(colp, rowp, zrs16)
    dis16, y = _prep(deg16, x0)
    s_sum = x0
    for _ in range(3):
        xl, y = _spmm_kernel(y, scolP, srowP, cnts, zrs, dis16)
        s_sum = s_sum + xl
    anc, pos, neg = _bpr_gather_kernel(s_sum, users, items, neg_items)
    return _loss(anc, pos, neg)[0, 0]


# speedup vs baseline: 6.2231x; 1.0098x over previous
import functools

import jax
import jax.numpy as jnp
from jax import lax
from jax.experimental import pallas as pl
from jax.experimental.pallas import tpu as pltpu
from jax.experimental.pallas import tpu_sc as plsc

N_U = 30000
N_I = 20000
NN = 50000
EE = 800000
DD = 64
BB = 4096

H = 25000
CH = 1568
HP = 16 * CH
K = 64
NB = 6
BLK = NB * K
NBLK = 131
PER_SUB = NBLK * BLK
EP = 16 * PER_SUB

CAP = 50688
CAPC = CAP // K
NCHT = 32 * CAPC
NSTG = 8

_mesh = plsc.VectorSubcoreMesh(
    core_axis_name="c", subcore_axis_name="s", num_cores=2, num_subcores=16)


def _f32(shape):
    return jax.ShapeDtypeStruct(shape, jnp.float32)


def _i32(shape):
    return jax.ShapeDtypeStruct(shape, jnp.int32)


@functools.partial(
    pl.kernel,
    out_type=(_i32((NCHT, K)), _i32((NCHT, K)), _i32((32, 16)),
              _f32((NN, 16))),
    mesh=_mesh,
    compiler_params=pltpu.CompilerParams(use_tc_tiling_on_sc=False,
                                         needs_layout_passes=False),
    scratch_types=[
        pltpu.VMEM((2, BLK), jnp.int32),
        pltpu.VMEM((2, BLK), jnp.int32),
        pltpu.VMEM((2, NSTG, K), jnp.int32),
        pltpu.VMEM((2, NSTG, K), jnp.int32),
        pltpu.VMEM((16,), jnp.int32),
        pltpu.VMEM_SHARED((HP, 16), jnp.float32),
        pltpu.VMEM((K, 16), jnp.float32),
        pltpu.SemaphoreType.DMA((2,)),
        pltpu.SemaphoreType.DMA((NSTG,)),
        pltpu.SemaphoreType.DMA((NSTG,)),
    ],
)
def _part_kernel(colp, rowp, zrs16, srowP, scolP, cnts, deg16, colb, rowb,
                 sstg, cstg, cntv, dacc, onesv, isem, fsem, dsem):
    c = lax.axis_index("c")
    s = lax.axis_index("s")
    w = c * 16 + s
    obase = w * CAPC
    c_lo = c * H
    ebase = s * PER_SUB
    iota = lax.iota(jnp.int32, 16)

    pltpu.async_copy(colp.at[pl.ds(ebase, BLK)], colb.at[0], isem.at[0])
    pltpu.async_copy(rowp.at[pl.ds(ebase, BLK)], rowb.at[0], isem.at[0])
    pltpu.sync_copy(zrs16.at[pl.ds(s * CH, CH)], dacc.at[pl.ds(s * CH, CH)])
    pat = jnp.where(iota == 0, jnp.float32(1.0), jnp.float32(0.0))
    for k in range(K):
        onesv[k, :] = pat
    plsc.subcore_barrier()

    def block(b, carry):
        wpos, ocnt, nfl_prev = carry
        p = jnp.bitwise_and(b, 1)
        base = ebase + b * BLK
        pltpu.make_async_copy(
            colp.at[pl.ds(base, BLK)], colb.at[p], isem.at[p]).wait()
        pltpu.make_async_copy(
            rowp.at[pl.ds(base, BLK)], rowb.at[p], isem.at[p]).wait()

        @pl.when(b + 1 < NBLK)
        def _():
            nbase = base + BLK
            pltpu.async_copy(colp.at[pl.ds(nbase, BLK)],
                             colb.at[1 - p], isem.at[1 - p])
            pltpu.async_copy(rowp.at[pl.ds(nbase, BLK)],
                             rowb.at[1 - p], isem.at[1 - p])
        pch = obase + ocnt - nfl_prev
        for j in range(NSTG - 1):
            @pl.when(j < nfl_prev)
            def _():
                pltpu.make_async_copy(
                    sstg.at[1 - p, j], srowP.at[pch + j], fsem.at[j]).wait()
                pltpu.make_async_copy(
                    cstg.at[1 - p, j], scolP.at[pch + j], fsem.at[j]).wait()
                pltpu.make_async_copy(
                    onesv, dacc.at[sstg.at[1 - p, j]], dsem.at[j]).wait()
        for j in range(BLK // 16):
            r = rowb[p, pl.ds(j * 16, 16)]
            cv = colb[p, pl.ds(j * 16, 16)]
            ok = (r >= c_lo) & (r < c_lo + H)
            oki = jnp.where(ok, jnp.int32(1), jnp.int32(0))
            off = wpos + plsc.cumsum(oki) - 1
            d0 = lax.shift_right_logical(off, 6)
            d1 = jnp.bitwise_and(off, 63)
            plsc.store_scatter(sstg.at[p], [d0, d1], r - c_lo, mask=ok)
            plsc.store_scatter(cstg.at[p], [d0, d1], cv, mask=ok)
            wpos = wpos + lax.reduce_sum(oki, axes=(0,))
        nfl = lax.shift_right_logical(wpos, 6)
        fch = obase + ocnt
        for j in range(NSTG - 1):
            @pl.when(j < nfl)
            def _():
                pltpu.async_copy(sstg.at[p, j], srowP.at[fch + j], fsem.at[j])
                pltpu.async_copy(cstg.at[p, j], scolP.at[fch + j], fsem.at[j])
                pltpu.async_copy(onesv, dacc.at[sstg.at[p, j]],
                                 dsem.at[j], add=True)
        for i in range(4):
            sl = pl.ds(i * 16, 16)
            sstg[1 - p, 0, sl] = sstg[p, nfl, sl]
            cstg[1 - p, 0, sl] = cstg[p, nfl, sl]
        return jnp.bitwise_and(wpos, 63), ocnt + nfl, nfl

    wpos, ocnt, nfl_last = lax.fori_loop(
        0, NBLK, block, (jnp.int32(0), jnp.int32(0), jnp.int32(0)))
    lastp = (NBLK - 1) & 1
    pch = obase + ocnt - nfl_last
    for j in range(NSTG - 1):
        @pl.when(j < nfl_last)
        def _():
            pltpu.make_async_copy(
                sstg.at[1 - lastp, j], srowP.at[pch + j], fsem.at[j]).wait()
            pltpu.make_async_copy(
                cstg.at[1 - lastp, j], scolP.at[pch + j], fsem.at[j]).wait()
            pltpu.make_async_copy(
                onesv, dacc.at[sstg.at[1 - lastp, j]], dsem.at[j]).wait()
    fp = 1 - lastp
    for i in range(4):
        idxs = i * 16 + iota
        m = idxs >= wpos
        sl = pl.ds(i * 16, 16)
        plsc.store_scatter(sstg.at[fp, 0], [idxs], jnp.full((16,), H,
                                                           jnp.int32), mask=m)
        plsc.store_scatter(cstg.at[fp, 0], [idxs], jnp.zeros((16,),
                                                            jnp.int32), mask=m)
        sstg[fp, 1, sl] = jnp.full((16,), H, jnp.int32)
        cstg[fp, 1, sl] = jnp.zeros((16,), jnp.int32)
    pltpu.sync_copy(sstg.at[fp, 0], srowP.at[obase + ocnt])
    pltpu.sync_copy(cstg.at[fp, 0], scolP.at[obase + ocnt])
    pltpu.sync_copy(onesv, dacc.at[sstg.at[fp, 0]], add=True)
    ocnt = ocnt + 1
    nblk6 = lax.shift_right_logical(ocnt * 43691, 18)
    npad = nblk6 * 6 + 6 - ocnt
    npad = jnp.where(npad == 6, 0, npad)
    for j in range(5):
        @pl.when(j < npad)
        def _():
            pb = obase + ocnt + j
            pltpu.sync_copy(sstg.at[fp, 1], srowP.at[pb])
            pltpu.sync_copy(cstg.at[fp, 1], scolP.at[pb])
    ocnt = ocnt + npad
    nblk = lax.shift_right_logical(ocnt * 43691, 18)
    cntv[...] = jnp.full((16,), 1, jnp.int32) * nblk
    pltpu.sync_copy(cntv, cnts.at[w])
    plsc.subcore_barrier()
    ob = jnp.minimum(s * CH, H - CH)
    pltpu.sync_copy(dacc.at[pl.ds(ob, CH)], deg16.at[pl.ds(c_lo + ob, CH)])


@functools.partial(
    pl.kernel,
    out_type=(_f32((NN, DD)), _f32((NN, DD))),
    mesh=_mesh,
    compiler_params=pltpu.CompilerParams(use_tc_tiling_on_sc=False,
                                         needs_layout_passes=False),
    scratch_types=[
        pltpu.VMEM_SHARED((HP, DD), jnp.float32),
        pltpu.VMEM((2, NB, K), jnp.int32),
        pltpu.VMEM((2, NB, K), jnp.int32),
        pltpu.VMEM((16,), jnp.int32),
        pltpu.VMEM((BLK, DD), jnp.float32),
        pltpu.VMEM((224, 16), jnp.float32),
        pltpu.SemaphoreType.DMA((2,)),
        pltpu.SemaphoreType.DMA((NB,)),
        pltpu.SemaphoreType.DMA((NB,)),
    ],
)
def _spmm_kernel(y, scolP, srowP, cnts, dis16, xl, ynext,
                 acc, colb, rowb, cntv, gbuf, dbuf, isem, gsem, ssem):
    c = lax.axis_index("c")
    s = lax.axis_index("s")
    c_lo = c * H
    w = c * 16 + s
    obase = w * CAPC
    zv = jnp.zeros((16,), jnp.float32)

    def zrow(rr, carry):
        for q in range(DD // 16):
            gbuf[rr, pl.ds(q * 16, 16)] = zv
        return carry

    lax.fori_loop(0, 224, zrow, 0)

    def zcp(t, carry):
        pltpu.sync_copy(gbuf.at[pl.ds(0, 224)],
                        acc.at[pl.ds(s * CH + t * 224, 224)])
        return carry

    lax.fori_loop(0, CH // 224, zcp, 0)
    pltpu.sync_copy(cnts.at[w], cntv)
    nblk = lax.reduce_max(cntv[...], axes=(0,))
    plsc.subcore_barrier()

    pltpu.async_copy(scolP.at[pl.ds(obase, NB)], colb.at[0], isem.at[0])
    pltpu.async_copy(srowP.at[pl.ds(obase, NB)], rowb.at[0], isem.at[0])

    def block(b, carry):
        p = jnp.bitwise_and(b, 1)
        base = obase + b * NB
        pltpu.make_async_copy(
            scolP.at[pl.ds(base, NB)], colb.at[p], isem.at[p]).wait()
        pltpu.make_async_copy(
            srowP.at[pl.ds(base, NB)], rowb.at[p], isem.at[p]).wait()

        @pl.when(b + 1 < nblk)
        def _():
            nbase = base + NB
            pltpu.async_copy(scolP.at[pl.ds(nbase, NB)],
                             colb.at[1 - p], isem.at[1 - p])
            pltpu.async_copy(srowP.at[pl.ds(nbase, NB)],
                             rowb.at[1 - p], isem.at[1 - p])
        @pl.when(b > 0)
        def _():
            for j in range(NB):
                pltpu.make_async_copy(
                    gbuf.at[pl.ds(j * K, K)], acc.at[rowb.at[1 - p, j]],
                    ssem.at[j]).wait()
        gd = [pltpu.async_copy(y.at[colb.at[p, j]],
                               gbuf.at[pl.ds(j * K, K)], gsem.at[j])
              for j in range(NB)]
        for j in range(NB):
            gd[j].wait()
            pltpu.async_copy(gbuf.at[pl.ds(j * K, K)], acc.at[rowb.at[p, j]],
                             ssem.at[j], add=True)
        return carry

    lax.fori_loop(0, nblk, block, 0)
    lastp = jnp.bitwise_and(nblk - 1, 1)
    for j in range(NB):
        pltpu.make_async_copy(
            gbuf.at[pl.ds(j * K, K)], acc.at[rowb.at[lastp, j]],
            ssem.at[j]).wait()
    plsc.subcore_barrier()
    ob = jnp.minimum(s * CH, H - CH)

    def subblk(t, carry):
        r0 = ob + t * 224
        pltpu.sync_copy(acc.at[pl.ds(r0, 224)], gbuf.at[pl.ds(0, 224)])
        pltpu.sync_copy(dis16.at[pl.ds(c_lo + r0, 224)], dbuf)

        def rowscale(rr, carry2):
            d = dbuf[rr, :]
            for q in range(DD // 16):
                sl = pl.ds(q * 16, 16)
                gbuf[rr, sl] = gbuf[rr, sl] * d
            return carry2

        lax.fori_loop(0, 224, rowscale, 0)
        pltpu.sync_copy(gbuf.at[pl.ds(0, 224)], xl.at[pl.ds(c_lo + r0, 224)])
        lax.fori_loop(0, 224, rowscale, 0)
        pltpu.sync_copy(gbuf.at[pl.ds(0, 224)],
                        ynext.at[pl.ds(c_lo + r0, 224)])
        return carry

    lax.fori_loop(0, CH // 224, subblk, 0)


@functools.partial(
    pl.kernel,
    out_type=(_f32((BB, DD)), _f32((BB, DD)), _f32((BB, DD))),
    mesh=_mesh,
    compiler_params=pltpu.CompilerParams(use_tc_tiling_on_sc=False, needs_layout_passes=False),
    scratch_types=[
        pltpu.VMEM((128,), jnp.int32),
        pltpu.VMEM((128, DD), jnp.float32),
        pltpu.SemaphoreType.DMA,
    ],
)
def _bpr_gather_kernel(S, users, items, negs, anc, pos, neg, idxv, buf, sem):
    c = lax.axis_index("c")
    s = lax.axis_index("s")
    base = (s * 2 + c) * 128

    pltpu.sync_copy(users.at[pl.ds(base, 128)], idxv)
    pltpu.async_copy(S.at[idxv], buf, sem).wait()
    pltpu.sync_copy(buf, anc.at[pl.ds(base, 128)])

    for src, dst in ((items, pos), (negs, neg)):
        pltpu.sync_copy(src.at[pl.ds(base, 128)], idxv)
        for j in range(8):
            idxv[pl.ds(j * 16, 16)] = idxv[pl.ds(j * 16, 16)] + N_U
        pltpu.async_copy(S.at[idxv], buf, sem).wait()
        pltpu.sync_copy(buf, dst.at[pl.ds(base, 128)])


_RB = 2000


def _prep_body(deg_ref, x0_ref, dis_ref, y0_ref):
    d = jnp.clip(lax.rsqrt(deg_ref[:, 0:1] + 1e-6), 0.0, 10.0)
    dis_ref[...] = jnp.broadcast_to(d, (_RB, 16))
    y0_ref[...] = d * x0_ref[...]


_prep = pl.pallas_call(
    _prep_body,
    grid=(NN // _RB,),
    in_specs=[
        pl.BlockSpec((_RB, 16), lambda i: (i, 0)),
        pl.BlockSpec((_RB, DD), lambda i: (i, 0)),
    ],
    out_specs=[
        pl.BlockSpec((_RB, 16), lambda i: (i, 0)),
        pl.BlockSpec((_RB, DD), lambda i: (i, 0)),
    ],
    out_shape=[_f32((NN, 16)), _f32((NN, DD))],
)


def _loss_body(a_ref, p_ref, n_ref, o_ref):
    a = a_ref[...]
    p = p_ref[...]
    n = n_ref[...]
    diff = jnp.sum(a * p, axis=-1) - jnp.sum(a * n, axis=-1)
    bpr = -jnp.sum(jnp.log(jax.nn.sigmoid(diff) + 1e-12)) / float(BB)
    reg = 0.5 * (jnp.sum(a * a) + jnp.sum(p * p) + jnp.sum(n * n)) / float(BB)
    o_ref[...] = (bpr + reg).reshape(1, 1)


_loss = pl.pallas_call(
    _loss_body,
    out_shape=_f32((1, 1)),
)


def kernel(user_emb, item_emb, edge_index, users, items, neg_items):
    x0 = jnp.concatenate([user_emb, item_emb], axis=0)
    pad = EP - EE
    colp = jnp.concatenate([edge_index[1], jnp.zeros((pad,), jnp.int32)])
    rowp = jnp.concatenate([edge_index[0], jnp.full((pad,), -1, jnp.int32)])
    zrs16 = jnp.zeros((HP, 16), jnp.float32)

    srowP, scolP, cnts, deg16 = _part_kernel(colp, rowp, zrs16)
    dis16, y = _prep(deg16, x0)
    s_sum = x0
    for _ in range(3):
        xl, y = _spmm_kernel(y, scolP, srowP, cnts, dis16)
        s_sum = s_sum + xl
    anc, pos, neg = _bpr_gather_kernel(s_sum, users, items, neg_items)
    return _loss(anc, pos, neg)[0, 0]
